# Initial kernel scaffold; baseline (speedup 1.0000x reference)
import jax
import jax.numpy as jnp
from jax.experimental import pallas as pl


def kernel(k_imu, d_imu, phi_imu, c_imu, k_theta_imu, d_theta_imu, phi_theta_imu, c_theta_imu, seq_len, time_steps_propogate_kinematics):
    raise NotImplementedError("write your pallas kernel here")


# speedup vs baseline: 28.3813x; 28.3813x over previous
import jax
import jax.numpy as jnp
from jax.experimental import pallas as pl
from jax.experimental.pallas import tpu as pltpu

_SEQ = 4096
_TST = 2048
_RL = 256
_NC = _SEQ // _RL
_OS = 8
_NG = _SEQ // _OS


def _imu_body(k_ref, d_ref, phi_ref, c_ref, kt_ref, dt_ref, phit_ref, ct_ref,
              out_ref, a_scr, w_scr, at_scr, wt_scr):
    a_scr[...] = d_ref[...] * -0.5
    w_scr[...] = jnp.sqrt(k_ref[...] * 4.0 - d_ref[...] * d_ref[...]) * 0.5
    at_scr[...] = dt_ref[...] * -0.5
    wt_scr[...] = jnp.sqrt(kt_ref[...] * 4.0 - dt_ref[...] * dt_ref[...]) * 0.5

    sub = jax.lax.broadcasted_iota(jnp.int32, (_OS, _RL), 0).astype(jnp.float32)
    lane = jax.lax.broadcasted_iota(jnp.int32, (_OS, _RL), 1).astype(jnp.float32)
    sml = sub - lane

    def group_body(j, carry):
        p0 = j * _OS
        c_lo = jnp.maximum(p0 - (_TST - 1), 0) // _RL
        c_hi = (p0 + _OS - 1) // _RL

        def chunk_body(c, acc):
            base = (p0 - c * _RL).astype(jnp.float32)
            t = base + sml
            valid = (t >= 0.0) & (t < float(_TST))
            a = a_scr[pl.ds(c, 1), :]
            w = w_scr[pl.ds(c, 1), :]
            ph = phi_ref[pl.ds(c, 1), :]
            cc = c_ref[pl.ds(c, 1), :]
            at = at_scr[pl.ds(c, 1), :]
            wt = wt_scr[pl.ds(c, 1), :]
            pht = phit_ref[pl.ds(c, 1), :]
            ct = ct_ref[pl.ds(c, 1), :]
            v = (cc * jnp.exp(a * t) * jnp.sin(t * w + ph)
                 + ct * jnp.exp(at * t) * jnp.sin(t * wt + pht))
            return acc + jnp.where(valid, v, 0.0)

        acc = jax.lax.fori_loop(c_lo, c_hi + 1, chunk_body,
                                jnp.zeros((_OS, _RL), jnp.float32))
        out_ref[pl.ds(j, 1), :] = jnp.sum(acc, axis=1).reshape(1, _OS)
        return carry

    jax.lax.fori_loop(0, _NG, group_body, 0)


def kernel(k_imu, d_imu, phi_imu, c_imu, k_theta_imu, d_theta_imu,
           phi_theta_imu, c_theta_imu, seq_len,
           time_steps_propogate_kinematics):
    shape2 = (_NC, _RL)
    args = [jnp.asarray(x, jnp.float32).reshape(shape2) for x in
            (k_imu, d_imu, phi_imu, c_imu, k_theta_imu, d_theta_imu,
             phi_theta_imu, c_theta_imu)]
    out = pl.pallas_call(
        _imu_body,
        out_shape=jax.ShapeDtypeStruct((_NG, _OS), jnp.float32),
        scratch_shapes=[pltpu.VMEM((_NC, _RL), jnp.float32)] * 4,
    )(*args)
    return out.reshape(1, _SEQ)


# speedup vs baseline: 34.2661x; 1.2073x over previous
import jax
import jax.numpy as jnp
from jax.experimental import pallas as pl
from jax.experimental.pallas import tpu as pltpu

_SEQ = 4096
_TST = 2048
_RL = 256
_NC = _SEQ // _RL
_OS = 16
_NG = _SEQ // _OS


def _imu_body(k_ref, d_ref, phi_ref, c_ref, kt_ref, dt_ref, phit_ref, ct_ref,
              out_ref, a_scr, w_scr, at_scr, wt_scr):
    a_scr[...] = d_ref[...] * -0.5
    w_scr[...] = jnp.sqrt(k_ref[...] * 4.0 - d_ref[...] * d_ref[...]) * 0.5
    at_scr[...] = dt_ref[...] * -0.5
    wt_scr[...] = jnp.sqrt(kt_ref[...] * 4.0 - dt_ref[...] * dt_ref[...]) * 0.5

    sub = jax.lax.broadcasted_iota(jnp.int32, (_OS, _RL), 0).astype(jnp.float32)
    lane = jax.lax.broadcasted_iota(jnp.int32, (_OS, _RL), 1).astype(jnp.float32)
    sml = sub - lane

    def group_body(j, carry):
        p0 = j * _OS
        c_lo = jnp.maximum(p0 - (_TST - 1), 0) // _RL
        c_hi = (p0 + _OS - 1) // _RL

        def chunk_body(c, acc):
            base = (p0 - c * _RL).astype(jnp.float32)
            t = base + sml
            valid = (t >= 0.0) & (t < float(_TST))
            a = a_scr[pl.ds(c, 1), :]
            w = w_scr[pl.ds(c, 1), :]
            ph = phi_ref[pl.ds(c, 1), :]
            cc = c_ref[pl.ds(c, 1), :]
            at = at_scr[pl.ds(c, 1), :]
            wt = wt_scr[pl.ds(c, 1), :]
            pht = phit_ref[pl.ds(c, 1), :]
            ct = ct_ref[pl.ds(c, 1), :]
            v = (cc * jnp.exp(a * t) * jnp.sin(t * w + ph)
                 + ct * jnp.exp(at * t) * jnp.sin(t * wt + pht))
            return acc + jnp.where(valid, v, 0.0)

        acc = jax.lax.fori_loop(c_lo, c_hi + 1, chunk_body,
                                jnp.zeros((_OS, _RL), jnp.float32))
        out_ref[pl.ds(j, 1), :] = jnp.sum(acc, axis=1).reshape(1, _OS)
        return carry

    jax.lax.fori_loop(0, _NG, group_body, 0)


def kernel(k_imu, d_imu, phi_imu, c_imu, k_theta_imu, d_theta_imu,
           phi_theta_imu, c_theta_imu, seq_len,
           time_steps_propogate_kinematics):
    shape2 = (_NC, _RL)
    args = [jnp.asarray(x, jnp.float32).reshape(shape2) for x in
            (k_imu, d_imu, phi_imu, c_imu, k_theta_imu, d_theta_imu,
             phi_theta_imu, c_theta_imu)]
    out = pl.pallas_call(
        _imu_body,
        out_shape=jax.ShapeDtypeStruct((_NG, _OS), jnp.float32),
        scratch_shapes=[pltpu.VMEM((_NC, _RL), jnp.float32)] * 4,
    )(*args)
    return out.reshape(1, _SEQ)


# speedup vs baseline: 38.2147x; 1.1152x over previous
import jax
import jax.numpy as jnp
from jax.experimental import pallas as pl
from jax.experimental.pallas import tpu as pltpu

_SEQ = 4096
_TST = 2048
_RL = 256
_NC = _SEQ // _RL
_OS = 32
_NG = _SEQ // _OS


def _imu_body(k_ref, d_ref, phi_ref, c_ref, kt_ref, dt_ref, phit_ref, ct_ref,
              out_ref, a_scr, w_scr, at_scr, wt_scr):
    a_scr[...] = d_ref[...] * -0.5
    w_scr[...] = jnp.sqrt(k_ref[...] * 4.0 - d_ref[...] * d_ref[...]) * 0.5
    at_scr[...] = dt_ref[...] * -0.5
    wt_scr[...] = jnp.sqrt(kt_ref[...] * 4.0 - dt_ref[...] * dt_ref[...]) * 0.5

    sub = jax.lax.broadcasted_iota(jnp.int32, (_OS, _RL), 0).astype(jnp.float32)
    lane = jax.lax.broadcasted_iota(jnp.int32, (_OS, _RL), 1).astype(jnp.float32)
    sml = sub - lane

    def group_body(j, carry):
        p0 = j * _OS
        c_lo = jnp.maximum(p0 - (_TST - 1), 0) // _RL
        c_hi = (p0 + _OS - 1) // _RL

        def chunk_body(c, acc):
            base = (p0 - c * _RL).astype(jnp.float32)
            t = base + sml
            valid = (t >= 0.0) & (t < float(_TST))
            a = a_scr[pl.ds(c, 1), :]
            w = w_scr[pl.ds(c, 1), :]
            ph = phi_ref[pl.ds(c, 1), :]
            cc = c_ref[pl.ds(c, 1), :]
            at = at_scr[pl.ds(c, 1), :]
            wt = wt_scr[pl.ds(c, 1), :]
            pht = phit_ref[pl.ds(c, 1), :]
            ct = ct_ref[pl.ds(c, 1), :]
            v = (cc * jnp.exp(a * t) * jnp.sin(t * w + ph)
                 + ct * jnp.exp(at * t) * jnp.sin(t * wt + pht))
            return acc + jnp.where(valid, v, 0.0)

        acc = jax.lax.fori_loop(c_lo, c_hi + 1, chunk_body,
                                jnp.zeros((_OS, _RL), jnp.float32))
        out_ref[pl.ds(j, 1), :] = jnp.sum(acc, axis=1).reshape(1, _OS)
        return carry

    jax.lax.fori_loop(0, _NG, group_body, 0)


def kernel(k_imu, d_imu, phi_imu, c_imu, k_theta_imu, d_theta_imu,
           phi_theta_imu, c_theta_imu, seq_len,
           time_steps_propogate_kinematics):
    shape2 = (_NC, _RL)
    args = [jnp.asarray(x, jnp.float32).reshape(shape2) for x in
            (k_imu, d_imu, phi_imu, c_imu, k_theta_imu, d_theta_imu,
             phi_theta_imu, c_theta_imu)]
    out = pl.pallas_call(
        _imu_body,
        out_shape=jax.ShapeDtypeStruct((_NG, _OS), jnp.float32),
        scratch_shapes=[pltpu.VMEM((_NC, _RL), jnp.float32)] * 4,
    )(*args)
    return out.reshape(1, _SEQ)


# speedup vs baseline: 40.2705x; 1.0538x over previous
import jax
import jax.numpy as jnp
from jax.experimental import pallas as pl
from jax.experimental.pallas import tpu as pltpu

_SEQ = 4096
_TST = 2048
_RL = 256
_NC = _SEQ // _RL
_OS = 64
_NG = _SEQ // _OS


def _imu_body(k_ref, d_ref, phi_ref, c_ref, kt_ref, dt_ref, phit_ref, ct_ref,
              out_ref, a_scr, w_scr, at_scr, wt_scr):
    a_scr[...] = d_ref[...] * -0.5
    w_scr[...] = jnp.sqrt(k_ref[...] * 4.0 - d_ref[...] * d_ref[...]) * 0.5
    at_scr[...] = dt_ref[...] * -0.5
    wt_scr[...] = jnp.sqrt(kt_ref[...] * 4.0 - dt_ref[...] * dt_ref[...]) * 0.5

    sub = jax.lax.broadcasted_iota(jnp.int32, (_OS, _RL), 0).astype(jnp.float32)
    lane = jax.lax.broadcasted_iota(jnp.int32, (_OS, _RL), 1).astype(jnp.float32)
    sml = sub - lane

    def group_body(j, carry):
        p0 = j * _OS
        c_lo = jnp.maximum(p0 - (_TST - 1), 0) // _RL
        c_hi = (p0 + _OS - 1) // _RL

        def chunk_body(c, acc):
            base = (p0 - c * _RL).astype(jnp.float32)
            t = base + sml
            valid = (t >= 0.0) & (t < float(_TST))
            a = a_scr[pl.ds(c, 1), :]
            w = w_scr[pl.ds(c, 1), :]
            ph = phi_ref[pl.ds(c, 1), :]
            cc = c_ref[pl.ds(c, 1), :]
            at = at_scr[pl.ds(c, 1), :]
            wt = wt_scr[pl.ds(c, 1), :]
            pht = phit_ref[pl.ds(c, 1), :]
            ct = ct_ref[pl.ds(c, 1), :]
            v = (cc * jnp.exp(a * t) * jnp.sin(t * w + ph)
                 + ct * jnp.exp(at * t) * jnp.sin(t * wt + pht))
            return acc + jnp.where(valid, v, 0.0)

        acc = jax.lax.fori_loop(c_lo, c_hi + 1, chunk_body,
                                jnp.zeros((_OS, _RL), jnp.float32))
        out_ref[pl.ds(j, 1), :] = jnp.sum(acc, axis=1).reshape(1, _OS)
        return carry

    jax.lax.fori_loop(0, _NG, group_body, 0)


def kernel(k_imu, d_imu, phi_imu, c_imu, k_theta_imu, d_theta_imu,
           phi_theta_imu, c_theta_imu, seq_len,
           time_steps_propogate_kinematics):
    shape2 = (_NC, _RL)
    args = [jnp.asarray(x, jnp.float32).reshape(shape2) for x in
            (k_imu, d_imu, phi_imu, c_imu, k_theta_imu, d_theta_imu,
             phi_theta_imu, c_theta_imu)]
    out = pl.pallas_call(
        _imu_body,
        out_shape=jax.ShapeDtypeStruct((_NG, _OS), jnp.float32),
        scratch_shapes=[pltpu.VMEM((_NC, _RL), jnp.float32)] * 4,
    )(*args)
    return out.reshape(1, _SEQ)


# speedup vs baseline: 115.4255x; 2.8663x over previous
import jax
import jax.numpy as jnp
from jax.experimental import pallas as pl
from jax.experimental.pallas import tpu as pltpu

_SEQ = 4096
_TST = 2048
_RL = 256
_NC = _SEQ // _RL
_OS = 64
_NG = _SEQ // _OS


_S2PI_HI = 6.28125
_S2PI_LO = 0.0019353071795864846
_SINV2PI = 0.15915494309189535
_SC0 = 0.9999999528369572
_SC1 = -0.16666629704656394
_SC2 = 0.008332868373268382
_SC3 = -0.00019819995093551526
_SC4 = 2.7117597258194404e-06
_SC5 = -2.0823799434799284e-08


def _fast_sin(theta):
    n = jnp.floor(theta * _SINV2PI + 0.5)
    r = (theta - n * _S2PI_HI) - n * _S2PI_LO
    r2 = r * r
    p = _SC4 + r2 * _SC5
    p = _SC3 + r2 * p
    p = _SC2 + r2 * p
    p = _SC1 + r2 * p
    p = _SC0 + r2 * p
    return r * p


def _imu_body(k_ref, d_ref, phi_ref, c_ref, kt_ref, dt_ref, phit_ref, ct_ref,
              out_ref, a_scr, w_scr, at_scr, wt_scr):
    a_scr[...] = d_ref[...] * -0.5
    w_scr[...] = jnp.sqrt(k_ref[...] * 4.0 - d_ref[...] * d_ref[...]) * 0.5
    at_scr[...] = dt_ref[...] * -0.5
    wt_scr[...] = jnp.sqrt(kt_ref[...] * 4.0 - dt_ref[...] * dt_ref[...]) * 0.5

    sub = jax.lax.broadcasted_iota(jnp.int32, (_OS, _RL), 0).astype(jnp.float32)
    lane = jax.lax.broadcasted_iota(jnp.int32, (_OS, _RL), 1).astype(jnp.float32)
    sml = sub - lane

    def group_body(j, carry):
        p0 = j * _OS
        c_lo = jnp.maximum(p0 - (_TST - 1), 0) // _RL
        c_hi = (p0 + _OS - 1) // _RL

        def chunk_body(c, acc):
            base = (p0 - c * _RL).astype(jnp.float32)
            t = base + sml
            valid = (t >= 0.0) & (t < float(_TST))
            a = a_scr[pl.ds(c, 1), :]
            w = w_scr[pl.ds(c, 1), :]
            ph = phi_ref[pl.ds(c, 1), :]
            cc = c_ref[pl.ds(c, 1), :]
            at = at_scr[pl.ds(c, 1), :]
            wt = wt_scr[pl.ds(c, 1), :]
            pht = phit_ref[pl.ds(c, 1), :]
            ct = ct_ref[pl.ds(c, 1), :]
            v = (cc * jnp.exp(a * t) * _fast_sin(t * w + ph)
                 + ct * jnp.exp(at * t) * _fast_sin(t * wt + pht))
            return acc + jnp.where(valid, v, 0.0)

        acc = jax.lax.fori_loop(c_lo, c_hi + 1, chunk_body,
                                jnp.zeros((_OS, _RL), jnp.float32))
        out_ref[pl.ds(j, 1), :] = jnp.sum(acc, axis=1).reshape(1, _OS)
        return carry

    jax.lax.fori_loop(0, _NG, group_body, 0)


def kernel(k_imu, d_imu, phi_imu, c_imu, k_theta_imu, d_theta_imu,
           phi_theta_imu, c_theta_imu, seq_len,
           time_steps_propogate_kinematics):
    shape2 = (_NC, _RL)
    args = [jnp.asarray(x, jnp.float32).reshape(shape2) for x in
            (k_imu, d_imu, phi_imu, c_imu, k_theta_imu, d_theta_imu,
             phi_theta_imu, c_theta_imu)]
    out = pl.pallas_call(
        _imu_body,
        out_shape=jax.ShapeDtypeStruct((_NG, _OS), jnp.float32),
        scratch_shapes=[pltpu.VMEM((_NC, _RL), jnp.float32)] * 4,
    )(*args)
    return out.reshape(1, _SEQ)


# speedup vs baseline: 173.6836x; 1.5047x over previous
import jax
import jax.numpy as jnp
from jax.experimental import pallas as pl
from jax.experimental.pallas import tpu as pltpu

_SEQ = 4096
_TST = 2048
_RL = 256
_NC = _SEQ // _RL
_OS = 64
_SL = 8
_NSLAB = _OS // _SL
_NG = _SEQ // _OS

_S2PI_HI = 6.28125
_S2PI_LO = 0.0019353071795864846
_SINV2PI = 0.15915494309189535
_SIN_C = (0.9999999528369572, -0.16666629704656394, 0.008332868373268382,
          -0.00019819995093551526, 2.7117597258194404e-06,
          -2.0823799434799284e-08)
_COS_C = (0.9999994009689195, -0.4999953021394909, 0.04166075139470328,
          -0.0013861784143072344, 2.4240032927225208e-05,
          -2.2132124788409868e-07)


def _reduce_2pi(theta):
    n = jnp.floor(theta * _SINV2PI + 0.5)
    return (theta - n * _S2PI_HI) - n * _S2PI_LO


def _poly_even(r2, coeffs):
    p = coeffs[-1]
    for c in coeffs[-2::-1]:
        p = c + r2 * p
    return p


def _fast_sin(theta):
    r = _reduce_2pi(theta)
    return r * _poly_even(r * r, _SIN_C)


def _fast_sincos(theta):
    r = _reduce_2pi(theta)
    r2 = r * r
    return r * _poly_even(r2, _SIN_C), _poly_even(r2, _COS_C)


def _imu_body(k_ref, d_ref, phi_ref, c_ref, kt_ref, dt_ref, phit_ref, ct_ref,
              out_ref, a_scr, w_scr, at_scr, wt_scr,
              s8_scr, c8_scr, e8_scr, s8t_scr, c8t_scr, e8t_scr):
    a_scr[...] = d_ref[...] * -0.5
    w_scr[...] = jnp.sqrt(k_ref[...] * 4.0 - d_ref[...] * d_ref[...]) * 0.5
    at_scr[...] = dt_ref[...] * -0.5
    wt_scr[...] = jnp.sqrt(kt_ref[...] * 4.0 - dt_ref[...] * dt_ref[...]) * 0.5
    s8, c8 = _fast_sincos(w_scr[...] * float(_SL))
    s8_scr[...] = s8
    c8_scr[...] = c8
    e8_scr[...] = jnp.exp(a_scr[...] * float(_SL))
    s8t, c8t = _fast_sincos(wt_scr[...] * float(_SL))
    s8t_scr[...] = s8t
    c8t_scr[...] = c8t
    e8t_scr[...] = jnp.exp(at_scr[...] * float(_SL))

    sub = jax.lax.broadcasted_iota(jnp.int32, (_SL, _RL), 0).astype(jnp.float32)
    lane = jax.lax.broadcasted_iota(jnp.int32, (_SL, _RL), 1).astype(jnp.float32)
    sml = sub - lane

    zeros = jnp.zeros((_SL, _RL), jnp.float32)

    def group_body(j, carry):
        p0 = j * _OS
        c_lo = jnp.maximum(p0 - (_TST - 1), 0) // _RL
        c_hi = (p0 + _OS - 1) // _RL

        def chunk_body(c, accs):
            base = (p0 - c * _RL).astype(jnp.float32)
            t0 = base + sml
            a = a_scr[pl.ds(c, 1), :]
            w = w_scr[pl.ds(c, 1), :]
            ph = phi_ref[pl.ds(c, 1), :]
            cc = c_ref[pl.ds(c, 1), :]
            at = at_scr[pl.ds(c, 1), :]
            wt = wt_scr[pl.ds(c, 1), :]
            pht = phit_ref[pl.ds(c, 1), :]
            ct = ct_ref[pl.ds(c, 1), :]
            s8 = s8_scr[pl.ds(c, 1), :]
            c8 = c8_scr[pl.ds(c, 1), :]
            e8 = e8_scr[pl.ds(c, 1), :]
            s8t = s8t_scr[pl.ds(c, 1), :]
            c8t = c8t_scr[pl.ds(c, 1), :]
            e8t = e8t_scr[pl.ds(c, 1), :]

            s, co = _fast_sincos(t0 * w + ph)
            e = cc * jnp.exp(a * t0)
            st, cot = _fast_sincos(t0 * wt + pht)
            et = ct * jnp.exp(at * t0)

            new_accs = []
            for v in range(_NSLAB):
                t = t0 + float(v * _SL)
                valid = (t >= 0.0) & (t < float(_TST))
                val = e * s + et * st
                new_accs.append(accs[v] + jnp.where(valid, val, 0.0))
                if v + 1 < _NSLAB:
                    s, co = s * c8 + co * s8, co * c8 - s * s8
                    e = e * e8
                    st, cot = st * c8t + cot * s8t, cot * c8t - st * s8t
                    et = et * e8t
            return tuple(new_accs)

        accs = jax.lax.fori_loop(c_lo, c_hi + 1, chunk_body, (zeros,) * _NSLAB)
        for v in range(_NSLAB):
            out_ref[pl.ds(j * _NSLAB + v, 1), :] = (
                jnp.sum(accs[v], axis=1).reshape(1, _SL))
        return carry

    jax.lax.fori_loop(0, _NG, group_body, 0)


def kernel(k_imu, d_imu, phi_imu, c_imu, k_theta_imu, d_theta_imu,
           phi_theta_imu, c_theta_imu, seq_len,
           time_steps_propogate_kinematics):
    shape2 = (_NC, _RL)
    args = [jnp.asarray(x, jnp.float32).reshape(shape2) for x in
            (k_imu, d_imu, phi_imu, c_imu, k_theta_imu, d_theta_imu,
             phi_theta_imu, c_theta_imu)]
    out = pl.pallas_call(
        _imu_body,
        out_shape=jax.ShapeDtypeStruct((_SEQ // _SL, _SL), jnp.float32),
        scratch_shapes=[pltpu.VMEM((_NC, _RL), jnp.float32)] * 10,
    )(*args)
    return out.reshape(1, _SEQ)
